# Initial kernel scaffold; baseline (speedup 1.0000x reference)
import functools

import jax
import jax.numpy as jnp
from jax import lax
from jax.experimental import pallas as pl
from jax.experimental.pallas import tpu as pltpu
from jax.experimental.pallas import tpu_sc as plsc

K = 8192
D = 256
B = 16384
L = 16
NC = 2
NS = 16
NW = NC * NS
B_PER_W = B // NW
CHUNK = 128
NCHUNK = B_PER_W // CHUNK


def _normalize_rows(buf, nrows):
    def row_body(r, carry):
        vals = [buf[r, pl.ds(L * i, L)] for i in range(D // L)]
        acc = vals[0] * vals[0]
        for i in range(1, D // L):
            acc = acc + vals[i] * vals[i]
        ss = jnp.sum(acc)
        ssv = lax.broadcast_in_dim(ss, (L,), ())
        yi = jnp.int32(0x5F3759DF) - (plsc.bitcast(ssv, jnp.int32) >> 1)
        y = plsc.bitcast(yi, jnp.float32)
        xh = ssv * 0.5
        y = y * (1.5 - xh * y * y)
        y = y * (1.5 - xh * y * y)
        inv = jnp.minimum(y, 1e12)
        for i in range(D // L):
            buf[r, pl.ds(L * i, L)] = vals[i] * inv
        return carry

    lax.fori_loop(0, nrows, row_body, 0)


mesh = plsc.VectorSubcoreMesh(core_axis_name="c", subcore_axis_name="s")


@functools.partial(
    pl.kernel,
    out_type=jax.ShapeDtypeStruct((B, D), jnp.float32),
    mesh=mesh,
    scratch_types=[
        pltpu.VMEM((NCHUNK, CHUNK), jnp.int32),
        pltpu.VMEM((CHUNK, D), jnp.float32),
        pltpu.SemaphoreType.DMA,
    ],
)
def _spherical_projection_sc(x_hbm, emb_hbm, out_hbm, idx_v, buf, sem):
    wid = lax.axis_index("s") * NC + lax.axis_index("c")
    pltpu.sync_copy(x_hbm.at[pl.ds(wid * NCHUNK, NCHUNK)], idx_v)
    row_base = wid * B_PER_W
    for j in range(NCHUNK):
        pltpu.async_copy(emb_hbm.at[idx_v.at[j]], buf, sem).wait()
        _normalize_rows(buf, CHUNK)
        pltpu.sync_copy(buf, out_hbm.at[pl.ds(row_base + j * CHUNK, CHUNK)])


def kernel(x, emb_weight):
    x2 = x.reshape(B // CHUNK, CHUNK)
    out = _spherical_projection_sc(x2, emb_weight)
    return out.reshape(x.shape[0], x.shape[1], D)


# speedup vs baseline: 1.4514x; 1.4514x over previous
import functools

import jax
import jax.numpy as jnp
from jax import lax
from jax.experimental import pallas as pl
from jax.experimental.pallas import tpu as pltpu
from jax.experimental.pallas import tpu_sc as plsc

K = 8192
D = 256
B = 16384
L = 16
NC = 2
NS = 16
NW = NC * NS
B_PER_W = B // NW
CHUNK = 128
NCHUNK = B_PER_W // CHUNK


def _normalize_rows(buf, nrows):
    def row_body(r, carry):
        vals = [buf[r, pl.ds(L * i, L)] for i in range(D // L)]
        acc = vals[0] * vals[0]
        for i in range(1, D // L):
            acc = acc + vals[i] * vals[i]
        ss = jnp.sum(acc)
        ssv = lax.broadcast_in_dim(ss, (L,), ())
        yi = jnp.int32(0x5F3759DF) - (plsc.bitcast(ssv, jnp.int32) >> 1)
        y = plsc.bitcast(yi, jnp.float32)
        xh = ssv * 0.5
        y = y * (1.5 - xh * y * y)
        y = y * (1.5 - xh * y * y)
        inv = jnp.minimum(y, 1e12)
        for i in range(D // L):
            buf[r, pl.ds(L * i, L)] = vals[i] * inv
        return carry

    lax.fori_loop(0, nrows, row_body, 0)


mesh = plsc.VectorSubcoreMesh(core_axis_name="c", subcore_axis_name="s")


@functools.partial(
    pl.kernel,
    out_type=jax.ShapeDtypeStruct((B, D), jnp.float32),
    mesh=mesh,
    scratch_types=[
        pltpu.VMEM((NCHUNK, CHUNK), jnp.int32),
        pltpu.VMEM((CHUNK, D), jnp.float32),
        pltpu.SemaphoreType.DMA,
    ],
    compiler_params=pltpu.CompilerParams(needs_layout_passes=False),
)
def _spherical_projection_sc(x_hbm, emb_hbm, out_hbm, idx_v, buf, sem):
    wid = lax.axis_index("s") * NC + lax.axis_index("c")
    pltpu.sync_copy(x_hbm.at[pl.ds(wid * NCHUNK, NCHUNK)], idx_v)
    row_base = wid * B_PER_W
    for j in range(NCHUNK):
        pltpu.async_copy(emb_hbm.at[idx_v.at[j]], buf, sem).wait()
        _normalize_rows(buf, CHUNK)
        pltpu.sync_copy(buf, out_hbm.at[pl.ds(row_base + j * CHUNK, CHUNK)])


def kernel(x, emb_weight):
    x2 = x.reshape(B // CHUNK, CHUNK)
    out = _spherical_projection_sc(x2, emb_weight)
    return out.reshape(x.shape[0], x.shape[1], D)


# speedup vs baseline: 2.0415x; 1.4066x over previous
import functools

import jax
import jax.numpy as jnp
from jax import lax
from jax.experimental import pallas as pl
from jax.experimental.pallas import tpu as pltpu
from jax.experimental.pallas import tpu_sc as plsc

K = 8192
D = 256
B = 16384
L = 16
NC = 2
NS = 16
NW = NC * NS
B_PER_W = B // NW
CHUNK = 128
NCHUNK = B_PER_W // CHUNK


def _normalize_one(buf, r):
    vals = [buf[r, pl.ds(L * i, L)] for i in range(D // L)]
    acc = vals[0] * vals[0]
    for i in range(1, D // L):
        acc = acc + vals[i] * vals[i]
    ss = jnp.sum(acc)
    ssv = lax.broadcast_in_dim(ss, (L,), ())
    yi = jnp.int32(0x5F3759DF) - (plsc.bitcast(ssv, jnp.int32) >> 1)
    y = plsc.bitcast(yi, jnp.float32)
    xh = ssv * 0.5
    y = y * (1.5 - xh * y * y)
    y = y * (1.5 - xh * y * y)
    inv = jnp.minimum(y, 1e12)
    for i in range(D // L):
        buf[r, pl.ds(L * i, L)] = vals[i] * inv


def _normalize_rows(buf, nrows, unroll=2):
    def row_body(i, carry):
        for u in range(unroll):
            _normalize_one(buf, i * unroll + u)
        return carry

    lax.fori_loop(0, nrows // unroll, row_body, 0)


mesh = plsc.VectorSubcoreMesh(core_axis_name="c", subcore_axis_name="s")


NBUF = 3


@functools.partial(
    pl.kernel,
    out_type=jax.ShapeDtypeStruct((B, D), jnp.float32),
    mesh=mesh,
    scratch_types=[
        pltpu.VMEM((NCHUNK, CHUNK), jnp.int32),
        pltpu.VMEM((NBUF, CHUNK, D), jnp.float32),
    ]
    + [pltpu.SemaphoreType.DMA] * (2 * NBUF),
    compiler_params=pltpu.CompilerParams(needs_layout_passes=False),
)
def _spherical_projection_sc(x_hbm, emb_hbm, out_hbm, idx_v, buf, *sems):
    gsem, ssem = sems[:NBUF], sems[NBUF:]
    wid = lax.axis_index("s") * NC + lax.axis_index("c")
    pltpu.sync_copy(x_hbm.at[pl.ds(wid * NCHUNK, NCHUNK)], idx_v)
    row_base = wid * B_PER_W

    def start_gather(j):
        b = j % NBUF
        return pltpu.async_copy(emb_hbm.at[idx_v.at[j]], buf.at[b], gsem[b])

    def start_scatter(j):
        b = j % NBUF
        return pltpu.async_copy(
            buf.at[b], out_hbm.at[pl.ds(row_base + j * CHUNK, CHUNK)], ssem[b])

    gd = {j: start_gather(j) for j in range(min(NBUF, NCHUNK))}
    sd = {}
    for j in range(NCHUNK):
        gd[j].wait()
        _normalize_rows(buf.at[j % NBUF], CHUNK)
        sd[j] = start_scatter(j)
        nxt = j + NBUF
        if nxt < NCHUNK:
            sd[j].wait()
            gd[nxt] = start_gather(nxt)
    for j in range(NCHUNK):
        if j in sd and (j + NBUF) >= NCHUNK:
            sd[j].wait()


def kernel(x, emb_weight):
    x2 = x.reshape(B // CHUNK, CHUNK)
    out = _spherical_projection_sc(x2, emb_weight)
    return out.reshape(x.shape[0], x.shape[1], D)
